# baseline (speedup 1.0000x reference)
import functools

import jax
import jax.numpy as jnp
from jax import lax
from jax.experimental import pallas as pl
from jax.experimental.pallas import tpu as pltpu
from jax.experimental.pallas import tpu_sc as plsc

B = 16384
F = 26
W = 32
GW = 128
NC = 2
NS = 16
NW = NC * NS
BPW = B // NW
C = 16
IPC = C * F
NCHUNK = BPW // C
L = 16

_mesh = plsc.VectorSubcoreMesh(core_axis_name="c", subcore_axis_name="s")


@functools.partial(
    pl.kernel,
    mesh=_mesh,
    compiler_params=pltpu.CompilerParams(
        use_tc_tiling_on_sc=False, needs_layout_passes=False
    ),
    out_type=jax.ShapeDtypeStruct((B, W), jnp.float32),
    scratch_types=[
        pltpu.VMEM((NCHUNK, IPC), jnp.int32),
        pltpu.VMEM((2, IPC), jnp.int32),
        pltpu.VMEM((2, IPC, GW), jnp.float32),
        pltpu.VMEM((2, C, W), jnp.float32),
        pltpu.SemaphoreType.DMA,
        pltpu.SemaphoreType.DMA,
        pltpu.SemaphoreType.DMA,
        pltpu.SemaphoreType.DMA,
    ],
)
def _embed_sum(x_hbm, emb_hbm, out_hbm, idx_v, row_v, rows_v, outs_v,
               sem_a, sem_b, sem_oa, sem_ob):
    wid = lax.axis_index("c") * NS + lax.axis_index("s")

    pltpu.sync_copy(x_hbm.at[wid], idx_v)

    sems = (sem_a, sem_b)
    osems = (sem_oa, sem_ob)
    iota = lax.iota(jnp.int32, L)

    def issue(chunk, buf):
        for k in range(IPC // L):
            row_v[buf, pl.ds(k * L, L)] = lax.shift_right_logical(
                idx_v[chunk, pl.ds(k * L, L)], 2
            )
        pltpu.async_copy(emb_hbm.at[row_v.at[buf]], rows_v.at[buf], sems[buf])

    def wait(buf):
        pltpu.make_async_copy(
            emb_hbm.at[pl.ds(0, IPC)], rows_v.at[buf], sems[buf]
        ).wait()

    def compute(chunk, buf):
        bufv = iota * 0 + buf
        chv = iota * 0 + chunk
        row_base = iota * F
        cbs = []
        for f in range(F):
            xi = plsc.load_gather(idx_v, [chv, row_base + f])
            cbs.append((xi & 3) * W)
        for c in range(W):
            acc = plsc.load_gather(rows_v, [bufv, row_base, cbs[0] + c])
            for f in range(1, F):
                acc = acc + plsc.load_gather(
                    rows_v, [bufv, row_base + f, cbs[f] + c]
                )
            plsc.store_scatter(outs_v, [bufv, iota, iota * 0 + c], acc)

    def out_wait(buf):
        pltpu.make_async_copy(
            emb_hbm.at[pl.ds(0, C), pl.ds(0, W)], outs_v.at[buf], osems[buf]
        ).wait()

    def out_issue(chunk, buf):
        pltpu.async_copy(
            outs_v.at[buf],
            out_hbm.at[pl.ds(wid * BPW + chunk * C, C)],
            osems[buf],
        )

    issue(0, 0)

    def body(i, carry):
        g = 2 * i
        issue(g + 1, 1)
        wait(0)

        @pl.when(g >= 2)
        def _():
            out_wait(0)

        compute(g, 0)
        out_issue(g, 0)

        @pl.when(g + 2 < NCHUNK)
        def _():
            issue(g + 2, 0)

        wait(1)

        @pl.when(g >= 2)
        def _():
            out_wait(1)

        compute(g + 1, 1)
        out_issue(g + 1, 1)
        return carry

    lax.fori_loop(0, NCHUNK // 2, body, 0)
    out_wait(0)
    out_wait(1)


def kernel(x, embeddings):
    x = x.astype(jnp.int32).reshape(NW, NCHUNK, IPC)
    emb = embeddings.reshape(-1, GW)
    return _embed_sum(x, emb)
